# baseline (speedup 1.0000x reference)
import functools

import jax
import jax.numpy as jnp
from jax import lax
from jax.experimental import pallas as pl
from jax.experimental.pallas import tpu as pltpu
from jax.experimental.pallas import tpu_sc as plsc

_NUM_CORES = 2
_NUM_SUBCORES = 16
_REP = 8


def kernel(inputs, pos_embeddings):
    batch, seq, dims = inputs.shape
    row = seq * dims
    pos = pos_embeddings[:seq, :].reshape(row)

    nw = _NUM_CORES * _NUM_SUBCORES
    b_per_w = batch // nw
    rep = _REP
    while b_per_w % rep:
        rep //= 2
    n_dma = b_per_w // rep

    mesh = plsc.VectorSubcoreMesh(
        core_axis_name="c",
        subcore_axis_name="s",
        num_cores=_NUM_CORES,
        num_subcores=_NUM_SUBCORES,
    )

    @functools.partial(
        pl.kernel,
        out_type=jax.ShapeDtypeStruct((batch * row,), jnp.float32),
        mesh=mesh,
        scratch_types=[
            pltpu.VMEM((rep * row,), jnp.float32),
            pltpu.SemaphoreType.DMA,
        ],
    )
    def run(pos_hbm, out_hbm, buf, sem_in):
        wid = lax.axis_index("s") * _NUM_CORES + lax.axis_index("c")
        base = wid * (b_per_w * row)
        fills = [
            pltpu.async_copy(pos_hbm, buf.at[pl.ds(r * row, row)], sem_in)
            for r in range(rep)
        ]
        for cp in fills:
            cp.wait()
        for j in range(n_dma):
            pltpu.sync_copy(buf, out_hbm.at[pl.ds(base + j * (rep * row), rep * row)])

    return run(pos).reshape(batch, seq, dims)


# speedup vs baseline: 1.0359x; 1.0359x over previous
import functools

import jax
import jax.numpy as jnp
from jax import lax
from jax.experimental import pallas as pl
from jax.experimental.pallas import tpu as pltpu
from jax.experimental.pallas import tpu_sc as plsc

_NUM_CORES = 2
_NUM_SUBCORES = 16
_REP = 8


def kernel(inputs, pos_embeddings):
    batch, seq, dims = inputs.shape
    row = seq * dims
    pos = pos_embeddings[:seq, :].reshape(row)

    nw = _NUM_CORES * _NUM_SUBCORES
    b_per_w = batch // nw
    rep = _REP
    while b_per_w % rep:
        rep //= 2
    n_dma = b_per_w // rep

    mesh = plsc.VectorSubcoreMesh(
        core_axis_name="c",
        subcore_axis_name="s",
        num_cores=_NUM_CORES,
        num_subcores=_NUM_SUBCORES,
    )

    @functools.partial(
        pl.kernel,
        out_type=jax.ShapeDtypeStruct((batch * row,), jnp.float32),
        mesh=mesh,
        scratch_types=[
            pltpu.VMEM((rep * row,), jnp.float32),
            pltpu.SemaphoreType.DMA,
        ],
    )
    def run(pos_hbm, out_hbm, buf, sem_in):
        wid = lax.axis_index("s") * _NUM_CORES + lax.axis_index("c")
        base = wid * (b_per_w * row)
        fills = [
            pltpu.async_copy(pos_hbm, buf.at[pl.ds(r * row, row)], sem_in)
            for r in range(rep)
        ]
        for cp in fills:
            cp.wait()
        for j in range(n_dma):
            shifted = lax.rem(j + wid, n_dma)
            pltpu.sync_copy(
                buf, out_hbm.at[pl.ds(base + shifted * (rep * row), rep * row)]
            )

    return run(pos).reshape(batch, seq, dims)
